# baseline (speedup 1.0000x reference)
import functools

import jax
import jax.numpy as jnp
from jax import lax
from jax.experimental import pallas as pl
from jax.experimental.pallas import tpu as pltpu
from jax.experimental.pallas import tpu_sc as plsc

NC = 2
NS = 16
NW = NC * NS
CL = 80
NBUF = 6
NBUF_SP = 2
DEPTH = 4


@functools.cache
def _build(n_total: int, n_chunks: int, d: int):
    mesh = plsc.VectorSubcoreMesh(core_axis_name="c", subcore_axis_name="s")
    per_w = n_total // NW
    j_lo = NBUF
    n_steady = ((n_chunks - DEPTH - j_lo) // NBUF) * NBUF
    assert n_steady >= 0 and NBUF % NBUF_SP == 0

    @functools.partial(
        pl.kernel,
        mesh=mesh,
        out_type=jax.ShapeDtypeStruct((n_total, d), jnp.float32),
        scratch_types=[
            pltpu.VMEM((n_chunks, CL), jnp.int32),
            pltpu.VMEM((NBUF, CL, d), jnp.float32),
            pltpu.VMEM_SHARED((NS, NBUF_SP, CL, d), jnp.float32),
            [pltpu.SemaphoreType.DMA] * NBUF,
            [pltpu.SemaphoreType.DMA] * NBUF_SP,
            [pltpu.SemaphoreType.DMA] * NBUF_SP,
        ],
    )
    def gather_kernel(
        idx_hbm, table_hbm, out_hbm, idx_v, rows_v, sp_sh, gsems, csems, osems
    ):
        sid = lax.axis_index("s")
        wid = sid * NC + lax.axis_index("c")
        base = wid * per_w
        sp_v = sp_sh.at[sid]

        pltpu.sync_copy(idx_hbm.at[wid], idx_v)

        def gather(j, rb):
            pltpu.async_copy(table_hbm.at[idx_v.at[j]], rows_v.at[rb], gsems[rb])

        def wait_gather(rb):
            pltpu.make_async_copy(
                table_hbm.at[idx_v.at[0]], rows_v.at[rb], gsems[rb]
            ).wait()

        def fwd(rb, sb):
            pltpu.async_copy(rows_v.at[rb], sp_v.at[sb], csems[sb])

        def wait_fwd(sb):
            pltpu.make_async_copy(rows_v.at[0], sp_v.at[sb], csems[sb]).wait()

        def drain(j, sb):
            pltpu.async_copy(
                sp_v.at[sb], out_hbm.at[pl.ds(base + j * CL, CL)], osems[sb]
            )

        def wait_drain(sb):
            pltpu.make_async_copy(
                sp_v.at[sb], out_hbm.at[pl.ds(base, CL)], osems[sb]
            ).wait()

        def step(j, b, steady=False):
            jn = j + DEPTH
            jp = j - (NBUF - DEPTH)
            rb_n = (b + DEPTH) % NBUF
            sb_p = (b + DEPTH) % NBUF_SP
            sb = b % NBUF_SP
            if steady or jp >= 0:
                wait_fwd(sb_p)
                drain(jp, sb_p)
            if steady or jn < n_chunks:
                gather(jn, rb_n)
            wait_gather(b)
            if steady or j - NBUF_SP >= 0:
                wait_drain(sb)
            fwd(b, sb)

        for j in range(DEPTH):
            gather(j, j % NBUF)
        for j in range(j_lo):
            step(j, j % NBUF)

        def body(io, carry):
            j0 = j_lo + io * NBUF
            for t in range(NBUF):
                step(j0 + t, t, steady=True)
            return carry

        lax.fori_loop(0, n_steady // NBUF, body, 0)

        for j in range(j_lo + n_steady, n_chunks):
            step(j, j % NBUF)

        for j in range(n_chunks - (NBUF - DEPTH), n_chunks):
            wait_fwd(j % NBUF_SP)
            drain(j, j % NBUF_SP)
        for j in range(n_chunks - NBUF_SP, n_chunks):
            wait_drain(j % NBUF_SP)

    return gather_kernel


def kernel(token_ids, W):
    b, l = token_ids.shape
    d = W.shape[1]
    n_total = b * l
    idx = token_ids.reshape(-1).astype(jnp.int32)
    n_chunks = n_total // (NW * CL)
    idx3 = idx.reshape(NW, n_chunks, CL)
    out = _build(n_total, n_chunks, d)(idx3, W)
    return out.reshape(b, l, d)


# speedup vs baseline: 1.0048x; 1.0048x over previous
import functools

import jax
import jax.numpy as jnp
from jax import lax
from jax.experimental import pallas as pl
from jax.experimental.pallas import tpu as pltpu
from jax.experimental.pallas import tpu_sc as plsc

NC = 2
NS = 16
NW = NC * NS
CL = 128
NP = 3


@functools.cache
def _build(n_total: int, n_chunks: int, d: int):
    mesh = plsc.VectorSubcoreMesh(core_axis_name="c", subcore_axis_name="s")
    per_w = n_total // NW
    assert n_chunks % 2 == 0
    n_pairs = n_chunks // 2
    p_lo = 1
    n_steady = ((n_pairs - 2 - p_lo) // NP) * NP
    assert n_steady >= 0

    @functools.partial(
        pl.kernel,
        mesh=mesh,
        out_type=jax.ShapeDtypeStruct((n_total, d), jnp.float32),
        scratch_types=[
            pltpu.VMEM((n_chunks, CL), jnp.int32),
            pltpu.VMEM((NP, 2 * CL, d), jnp.float32),
            [[pltpu.SemaphoreType.DMA] * 2] * NP,
            [pltpu.SemaphoreType.DMA] * NP,
        ],
    )
    def gather_kernel(idx_hbm, table_hbm, out_hbm, idx_v, rows_v, gsems, osems):
        wid = lax.axis_index("s") * NC + lax.axis_index("c")
        base = wid * per_w

        pltpu.sync_copy(idx_hbm.at[wid], idx_v)

        def gather_pair(p, bp):
            for h in (0, 1):
                pltpu.async_copy(
                    table_hbm.at[idx_v.at[2 * p + h]],
                    rows_v.at[bp, pl.ds(h * CL, CL)],
                    gsems[bp][h],
                )

        def wait_gather_pair(bp):
            for h in (0, 1):
                pltpu.make_async_copy(
                    table_hbm.at[idx_v.at[0]],
                    rows_v.at[bp, pl.ds(h * CL, CL)],
                    gsems[bp][h],
                ).wait()

        def write_pair(p, bp):
            pltpu.async_copy(
                rows_v.at[bp],
                out_hbm.at[pl.ds(base + p * 2 * CL, 2 * CL)],
                osems[bp],
            )

        def wait_write(bp):
            pltpu.make_async_copy(
                rows_v.at[bp], out_hbm.at[pl.ds(base, 2 * CL)], osems[bp]
            ).wait()

        def step(p, b):
            pn = p + 2
            bp = pn % NP
            if p - 1 >= 0:
                wait_write(bp)
            if pn < n_pairs:
                gather_pair(pn, bp)
            wait_gather_pair(b)
            write_pair(p, b)

        gather_pair(0, 0)
        gather_pair(1, 1)
        for p in range(p_lo):
            step(p, p % NP)

        def body(io, carry):
            p0 = p_lo + io * NP
            for t in range(NP):
                p = p0 + t
                b = (p_lo + t) % NP
                bp = (p_lo + t + 2) % NP
                wait_write(bp)
                gather_pair(p + 2, bp)
                wait_gather_pair(b)
                write_pair(p, b)
            return carry

        lax.fori_loop(0, n_steady // NP, body, 0)

        for p in range(p_lo + n_steady, n_pairs):
            step(p, p % NP)

        wait_write((n_pairs - 1) % NP)

    return gather_kernel


def kernel(token_ids, W):
    b, l = token_ids.shape
    d = W.shape[1]
    n_total = b * l
    idx = token_ids.reshape(-1).astype(jnp.int32)
    n_chunks = n_total // (NW * CL)
    idx3 = idx.reshape(NW, n_chunks, CL)
    out = _build(n_total, n_chunks, d)(idx3, W)
    return out.reshape(b, l, d)


# speedup vs baseline: 1.0056x; 1.0008x over previous
import functools

import jax
import jax.numpy as jnp
from jax import lax
from jax.experimental import pallas as pl
from jax.experimental.pallas import tpu as pltpu
from jax.experimental.pallas import tpu_sc as plsc

NC = 2
NS = 16
NW = NC * NS
CL = 128
NBUF = 6
DEPTH = 4


@functools.cache
def _build(n_total: int, n_chunks: int, d: int):
    mesh = plsc.VectorSubcoreMesh(core_axis_name="c", subcore_axis_name="s")
    per_w = n_total // NW
    j_lo = NBUF - DEPTH
    n_steady = ((n_chunks - DEPTH - j_lo) // NBUF) * NBUF
    assert n_steady >= 0

    @functools.partial(
        pl.kernel,
        mesh=mesh,
        out_type=jax.ShapeDtypeStruct((n_total, d), jnp.float32),
        scratch_types=[
            pltpu.VMEM((n_chunks, CL), jnp.int32),
            pltpu.VMEM((NBUF, CL, d), jnp.float32),
            [pltpu.SemaphoreType.DMA] * NBUF,
            [pltpu.SemaphoreType.DMA] * NBUF,
        ],
    )
    def gather_kernel(idx_hbm, table_hbm, out_hbm, idx_v, rows_v, gsems, osems):
        wid = lax.axis_index("s") * NC + lax.axis_index("c")
        base = wid * per_w

        pltpu.sync_copy(idx_hbm.at[wid], idx_v)

        def gather(j, b):
            pltpu.async_copy(table_hbm.at[idx_v.at[j]], rows_v.at[b], gsems[b])

        def wait_gather(b):
            pltpu.make_async_copy(
                table_hbm.at[idx_v.at[0]], rows_v.at[b], gsems[b]
            ).wait()

        def copy_out(j, b):
            pltpu.async_copy(
                rows_v.at[b], out_hbm.at[pl.ds(base + j * CL, CL)], osems[b]
            )

        def wait_out(b):
            pltpu.make_async_copy(
                rows_v.at[b], out_hbm.at[pl.ds(base, CL)], osems[b]
            ).wait()

        def step(j, b):
            jn = j + DEPTH
            bp = jn % NBUF
            if jn - NBUF >= 0:
                wait_out(bp)
            if jn < n_chunks:
                gather(jn, bp)
            wait_gather(b)
            copy_out(j, b)

        for j in range(DEPTH):
            gather(j, j % NBUF)
        for j in range(j_lo):
            step(j, j % NBUF)

        def body(io, carry):
            j0 = j_lo + io * NBUF
            for t in range(NBUF):
                j = j0 + t
                b = (j_lo + t) % NBUF
                jn = j + DEPTH
                bp = (j_lo + t + DEPTH) % NBUF
                wait_out(bp)
                gather(jn, bp)
                wait_gather(b)
                copy_out(j, b)
            return carry

        lax.fori_loop(0, n_steady // NBUF, body, 0)

        for j in range(j_lo + n_steady, n_chunks):
            step(j, j % NBUF)

        for j in range(n_chunks - NBUF + DEPTH, n_chunks):
            wait_out(j % NBUF)

    return gather_kernel


def kernel(token_ids, W):
    b, l = token_ids.shape
    d = W.shape[1]
    n_total = b * l
    idx = token_ids.reshape(-1).astype(jnp.int32)
    n_chunks = n_total // (NW * CL)
    idx3 = idx.reshape(NW, n_chunks, CL)
    out = _build(n_total, n_chunks, d)(idx3, W)
    return out.reshape(b, l, d)
